# baseline (speedup 1.0000x reference)
import dataclasses
import functools

import numpy as np
import jax
import jax.numpy as jnp
from jax import lax
from jax.experimental import pallas as pl
from jax.experimental.pallas import tpu as pltpu
from jax.experimental.pallas import tpu_sc as plsc

NUM_WORKERS = 32
P = 8
IDXW = 96


def _transpose_body(cv_ref, out_ref):
    out_ref[...] = cv_ref[0].T


def _meta_body(h, bands, pts_ref, m_ref, j0_ref, j1_ref, w0_ref, w1_ref, y_ref):
    pts = pts_ref[...]
    m = m_ref[...]
    fp = (pts[:, 0:1] * m[0:1, :] + pts[:, 1:2] * m[1:2, :]
          + pts[:, 2:3] * m[2:3, :])
    nb = fp.shape[0]
    col = lax.broadcasted_iota(jnp.int32, (nb, bands), 1)
    is_sin = ((col // 3) % 2) == 0
    y = jnp.where(is_sin, jnp.sin(fp), jnp.cos(fp))
    iy = (y + 1.0) * ((h - 1) * 0.5)
    i0f = jnp.floor(iy)
    fr = iy - i0f
    i1f = i0f + 1.0
    v0 = ((i0f >= 0.0) & (i0f <= h - 1.0)).astype(jnp.float32)
    v1 = ((i1f >= 0.0) & (i1f <= h - 1.0)).astype(jnp.float32)
    i0 = jnp.clip(i0f, 0.0, h - 1.0).astype(jnp.int32)
    i1 = jnp.clip(i1f, 0.0, h - 1.0).astype(jnp.int32)
    j0_ref[...] = i0 + col * h
    j1_ref[...] = i1 + col * h
    w0_ref[...] = (1.0 - fr) * v0
    w1_ref[...] = fr * v1
    y_ref[...] = y


def _sc_body(npw, nchunks, bands, c, cvt_hbm, j0_hbm, j1_hbm, meta_hbm,
             out_hbm, jv0, jv1, m_v, rv00, rv01, rv02, rv10, rv11, rv12,
             o_v, semg):
    ncores = 2
    wid = lax.axis_index("s") * ncores + lax.axis_index("c")
    pt_base = wid * npw
    ngather = (P * bands) // IDXW
    cols = bands * c
    tpc = P * bands
    mstride = tpc * 3 + 16
    lane = lax.iota(jnp.int32, 16)
    lane_cols = [lane * bands + k * 16 * bands for k in range(c // 16)]
    rv = ((rv00, rv01, rv02), (rv10, rv11, rv12))
    jv = (jv0, jv1)

    @pl.loop(0, nchunks)
    def _chunk(ci):
        t0 = (pt_base + ci * P) * bands
        for t in range(2):
            for g in range(ngather):
                pltpu.sync_copy((j0_hbm, j1_hbm)[t].at[pl.ds(t0 + g * IDXW,
                                                             IDXW)],
                                jv[t].at[g])
        gch = (pt_base // P) + ci
        pltpu.sync_copy(meta_hbm.at[pl.ds(gch * mstride, mstride)], m_v)
        handles = []
        for t in range(2):
            for g in range(ngather):
                handles.append(pltpu.async_copy(
                    cvt_hbm.at[jv[t].at[g]], rv[t][g], semg.at[t]))
        for hnd in handles:
            hnd.wait()

        for g in range(ngather):
            @pl.loop(0, IDXW)
            def _row(r, g=g):
                t = g * IDXW + r
                n = t // bands
                b = t % bands
                tv = plsc.load_gather(m_v,
                                      [jnp.full((16,), t * 3, jnp.int32)
                                       + lane])
                w0s = tv[0]
                w1s = tv[1]
                ys = tv[2]
                flat = jnp.full((16,), n * cols + b, jnp.int32)
                for k in range(c // 16):
                    gg0 = rv[0][g][r, pl.ds(k * 16, 16)]
                    gg1 = rv[1][g][r, pl.ds(k * 16, 16)]
                    val = gg0 * w0s + gg1 * w1s + ys
                    plsc.store_scatter(o_v, [flat + lane_cols[k]], val)

        pltpu.sync_copy(o_v,
                        out_hbm.at[pl.ds((pt_base + ci * P) * cols, P * cols)])


def kernel(points, scale, freqs, cv):
    n = points.shape[0]
    f = freqs.shape[0]
    bands = f * 2 * 3
    c = cv.shape[1]
    h = cv.shape[2]
    cols = bands * c
    assert n % (NUM_WORKERS * P) == 0 and c % 16 == 0 and (P * bands) % IDXW == 0

    fidx = np.arange(bands) // (2 * 3)
    dsel = np.arange(bands) % 3
    onehot = jnp.asarray((dsel[None, :] == np.arange(3)[:, None]).astype(np.float32))
    m = onehot * (freqs[fidx][None, :] / scale)

    cvt = pl.pallas_call(
        _transpose_body,
        grid=(bands,),
        in_specs=[pl.BlockSpec((1, c, h), lambda b: (b, 0, 0))],
        out_specs=pl.BlockSpec((h, c), lambda b: (b, 0)),
        out_shape=jax.ShapeDtypeStruct((bands * h, c), jnp.float32),
    )(cv.reshape(bands, c, h))

    nb = 2048
    j0, j1, w0, w1, y = pl.pallas_call(
        functools.partial(_meta_body, h, bands),
        grid=(n // nb,),
        in_specs=[pl.BlockSpec((nb, 3), lambda i: (i, 0)),
                  pl.BlockSpec((3, bands), lambda i: (0, 0))],
        out_specs=[pl.BlockSpec((nb, bands), lambda i: (i, 0))] * 5,
        out_shape=[jax.ShapeDtypeStruct((n, bands), jnp.int32),
                   jax.ShapeDtypeStruct((n, bands), jnp.int32),
                   jax.ShapeDtypeStruct((n, bands), jnp.float32),
                   jax.ShapeDtypeStruct((n, bands), jnp.float32),
                   jax.ShapeDtypeStruct((n, bands), jnp.float32)],
    )(points, m)

    npw = n // NUM_WORKERS
    nchunks = npw // P
    ngather = (P * bands) // IDXW
    mesh = plsc.VectorSubcoreMesh(core_axis_name="c", subcore_axis_name="s")
    cp = pltpu.CompilerParams()
    if "needs_layout_passes" in pltpu.CompilerParams.__dataclass_fields__:
        cp = dataclasses.replace(cp, needs_layout_passes=False)
    if "use_tc_tiling_on_sc" in pltpu.CompilerParams.__dataclass_fields__:
        cp = dataclasses.replace(cp, use_tc_tiling_on_sc=False)
    tpc = P * bands
    sc = pl.kernel(
        functools.partial(_sc_body, npw, nchunks, bands, c),
        compiler_params=cp,
        out_type=jax.ShapeDtypeStruct((n * cols,), jnp.float32),
        mesh=mesh,
        scratch_types=(
            [pltpu.VMEM((ngather, IDXW), jnp.int32)] * 2
            + [pltpu.VMEM((tpc * 3 + 16,), jnp.float32)]
            + [pltpu.VMEM((IDXW, c), jnp.float32)] * (2 * ngather)
            + [pltpu.VMEM((P * cols,), jnp.float32),
               pltpu.SemaphoreType.DMA((2,))]
        ),
    )
    meta = jnp.stack([w0, w1, y], axis=-1).reshape(n // P, tpc * 3)
    meta = jnp.pad(meta, ((0, 0), (0, 16))).reshape(-1)
    out = sc(cvt, j0.reshape(-1), j1.reshape(-1), meta)
    return out.reshape(n, cols)


# speedup vs baseline: 1.0579x; 1.0579x over previous
import dataclasses
import functools

import numpy as np
import jax
import jax.numpy as jnp
from jax import lax
from jax.experimental import pallas as pl
from jax.experimental.pallas import tpu as pltpu
from jax.experimental.pallas import tpu_sc as plsc

NUM_WORKERS = 32
P = 16
IDXW = 96


def _transpose_body(cv_ref, out_ref):
    out_ref[...] = cv_ref[0].T


def _meta_body(h, bands, pts_ref, m_ref, j0_ref, j1_ref, w0_ref, w1_ref, y_ref):
    pts = pts_ref[...]
    m = m_ref[...]
    fp = (pts[:, 0:1] * m[0:1, :] + pts[:, 1:2] * m[1:2, :]
          + pts[:, 2:3] * m[2:3, :])
    nb = fp.shape[0]
    col = lax.broadcasted_iota(jnp.int32, (nb, bands), 1)
    is_sin = ((col // 3) % 2) == 0
    y = jnp.where(is_sin, jnp.sin(fp), jnp.cos(fp))
    iy = (y + 1.0) * ((h - 1) * 0.5)
    i0f = jnp.floor(iy)
    fr = iy - i0f
    i1f = i0f + 1.0
    v0 = ((i0f >= 0.0) & (i0f <= h - 1.0)).astype(jnp.float32)
    v1 = ((i1f >= 0.0) & (i1f <= h - 1.0)).astype(jnp.float32)
    i0 = jnp.clip(i0f, 0.0, h - 1.0).astype(jnp.int32)
    i1 = jnp.clip(i1f, 0.0, h - 1.0).astype(jnp.int32)
    j0_ref[...] = i0 + col * h
    j1_ref[...] = i1 + col * h
    w0_ref[...] = (1.0 - fr) * v0
    w1_ref[...] = fr * v1
    y_ref[...] = y


def _sc_body(npw, nchunks, bands, c, cvt_hbm, jidx_hbm, meta_hbm,
             out_hbm, jv, m_v, *rest):
    ncores = 2
    ngather = (P * bands) // IDXW
    rv = (rest[:ngather], rest[ngather:2 * ngather])
    o_v, semg = rest[2 * ngather], rest[2 * ngather + 1]
    wid = lax.axis_index("s") * ncores + lax.axis_index("c")
    pt_base = wid * npw
    cols = bands * c
    tpc = P * bands
    mstride = tpc * 3 + 16
    lane = lax.iota(jnp.int32, 16)
    lane_cols = [lane * bands + k * 16 * bands for k in range(c // 16)]

    @pl.loop(0, nchunks)
    def _chunk(ci):
        gch = (pt_base // P) + ci
        pltpu.sync_copy(jidx_hbm.at[pl.ds(gch * 16, 16)], jv)
        pltpu.sync_copy(meta_hbm.at[pl.ds(gch * mstride, mstride)], m_v)
        handles = []
        for t in range(2):
            for g in range(ngather):
                handles.append(pltpu.async_copy(
                    cvt_hbm.at[jv.at[t * ngather + g]], rv[t][g],
                    semg.at[t]))
        for hnd in handles:
            hnd.wait()

        for g in range(ngather):
            @pl.loop(0, IDXW)
            def _row(r, g=g):
                t = g * IDXW + r
                n = t // bands
                b = t % bands
                tv = plsc.load_gather(m_v,
                                      [jnp.full((16,), t * 3, jnp.int32)
                                       + lane])
                w0s = tv[0]
                w1s = tv[1]
                ys = tv[2]
                flat = jnp.full((16,), n * cols + b, jnp.int32)
                for k in range(c // 16):
                    gg0 = rv[0][g][r, pl.ds(k * 16, 16)]
                    gg1 = rv[1][g][r, pl.ds(k * 16, 16)]
                    val = gg0 * w0s + gg1 * w1s + ys
                    plsc.store_scatter(o_v, [flat + lane_cols[k]], val)

        pltpu.sync_copy(o_v,
                        out_hbm.at[pl.ds((pt_base + ci * P) * cols, P * cols)])


def kernel(points, scale, freqs, cv):
    n = points.shape[0]
    f = freqs.shape[0]
    bands = f * 2 * 3
    c = cv.shape[1]
    h = cv.shape[2]
    cols = bands * c
    assert n % (NUM_WORKERS * P) == 0 and c % 16 == 0 and (P * bands) % IDXW == 0

    fidx = np.arange(bands) // (2 * 3)
    dsel = np.arange(bands) % 3
    onehot = jnp.asarray((dsel[None, :] == np.arange(3)[:, None]).astype(np.float32))
    m = onehot * (freqs[fidx][None, :] / scale)

    cvt = pl.pallas_call(
        _transpose_body,
        grid=(bands,),
        in_specs=[pl.BlockSpec((1, c, h), lambda b: (b, 0, 0))],
        out_specs=pl.BlockSpec((h, c), lambda b: (b, 0)),
        out_shape=jax.ShapeDtypeStruct((bands * h, c), jnp.float32),
    )(cv.reshape(bands, c, h))

    nb = 2048
    j0, j1, w0, w1, y = pl.pallas_call(
        functools.partial(_meta_body, h, bands),
        grid=(n // nb,),
        in_specs=[pl.BlockSpec((nb, 3), lambda i: (i, 0)),
                  pl.BlockSpec((3, bands), lambda i: (0, 0))],
        out_specs=[pl.BlockSpec((nb, bands), lambda i: (i, 0))] * 5,
        out_shape=[jax.ShapeDtypeStruct((n, bands), jnp.int32),
                   jax.ShapeDtypeStruct((n, bands), jnp.int32),
                   jax.ShapeDtypeStruct((n, bands), jnp.float32),
                   jax.ShapeDtypeStruct((n, bands), jnp.float32),
                   jax.ShapeDtypeStruct((n, bands), jnp.float32)],
    )(points, m)

    npw = n // NUM_WORKERS
    nchunks = npw // P
    ngather = (P * bands) // IDXW
    mesh = plsc.VectorSubcoreMesh(core_axis_name="c", subcore_axis_name="s")
    cp = pltpu.CompilerParams()
    if "needs_layout_passes" in pltpu.CompilerParams.__dataclass_fields__:
        cp = dataclasses.replace(cp, needs_layout_passes=False)
    if "use_tc_tiling_on_sc" in pltpu.CompilerParams.__dataclass_fields__:
        cp = dataclasses.replace(cp, use_tc_tiling_on_sc=False)
    tpc = P * bands
    sc = pl.kernel(
        functools.partial(_sc_body, npw, nchunks, bands, c),
        compiler_params=cp,
        out_type=jax.ShapeDtypeStruct((n * cols,), jnp.float32),
        mesh=mesh,
        scratch_types=(
            [pltpu.VMEM((16, IDXW), jnp.int32),
             pltpu.VMEM((tpc * 3 + 16,), jnp.float32)]
            + [pltpu.VMEM((IDXW, c), jnp.float32)] * (2 * ngather)
            + [pltpu.VMEM((P * cols,), jnp.float32),
               pltpu.SemaphoreType.DMA((2,))]
        ),
    )
    nct = n // P
    jblk = jnp.concatenate(
        [j0.reshape(nct, ngather, IDXW), j1.reshape(nct, ngather, IDXW),
         jnp.zeros((nct, 16 - 2 * ngather, IDXW), jnp.int32)], axis=1)
    meta = jnp.stack([w0, w1, y], axis=-1).reshape(nct, tpc * 3)
    meta = jnp.pad(meta, ((0, 0), (0, 16))).reshape(-1)
    out = sc(cvt, jblk.reshape(nct * 16, IDXW), meta)
    return out.reshape(n, cols)


# speedup vs baseline: 1.0802x; 1.0211x over previous
import dataclasses
import functools

import numpy as np
import jax
import jax.numpy as jnp
from jax import lax
from jax.experimental import pallas as pl
from jax.experimental.pallas import tpu as pltpu
from jax.experimental.pallas import tpu_sc as plsc

NUM_WORKERS = 32
P = 8
IDXW = 96
NSLOT = 2


def _transpose_body(cv_ref, out_ref):
    out_ref[...] = cv_ref[0].T


def _meta_body(h, bands, pts_ref, m_ref, j0_ref, j1_ref, w0_ref, w1_ref, y_ref):
    pts = pts_ref[...]
    m = m_ref[...]
    fp = (pts[:, 0:1] * m[0:1, :] + pts[:, 1:2] * m[1:2, :]
          + pts[:, 2:3] * m[2:3, :])
    nb = fp.shape[0]
    col = lax.broadcasted_iota(jnp.int32, (nb, bands), 1)
    is_sin = ((col // 3) % 2) == 0
    y = jnp.where(is_sin, jnp.sin(fp), jnp.cos(fp))
    iy = (y + 1.0) * ((h - 1) * 0.5)
    i0f = jnp.floor(iy)
    fr = iy - i0f
    i1f = i0f + 1.0
    v0 = ((i0f >= 0.0) & (i0f <= h - 1.0)).astype(jnp.float32)
    v1 = ((i1f >= 0.0) & (i1f <= h - 1.0)).astype(jnp.float32)
    i0 = jnp.clip(i0f, 0.0, h - 1.0).astype(jnp.int32)
    i1 = jnp.clip(i1f, 0.0, h - 1.0).astype(jnp.int32)
    j0_ref[...] = i0 + col * h
    j1_ref[...] = i1 + col * h
    w0_ref[...] = (1.0 - fr) * v0
    w1_ref[...] = fr * v1
    y_ref[...] = y


def _sc_body(npw, nchunks, bands, c, cvt_hbm, jidx_hbm, meta_hbm,
             out_hbm, *rest):
    ncores = 2
    ngather = (P * bands) // IDXW
    nrows = 2 * ngather + (-2 * ngather) % 8
    per = 2 + 2 * ngather + 1
    slots = [rest[s * per:(s + 1) * per] for s in range(NSLOT)]
    semg = rest[NSLOT * per]
    wid = lax.axis_index("s") * ncores + lax.axis_index("c")
    pt_base = wid * npw
    cols = bands * c
    tpc = P * bands
    mstride = tpc * 3 + 16
    lane = lax.iota(jnp.int32, 16)
    lane_cols = [lane * bands + k * 16 * bands for k in range(c // 16)]

    def stage(ci, s):
        jv, m_v = slots[s][0], slots[s][1]
        rv = (slots[s][2:2 + ngather], slots[s][2 + ngather:2 + 2 * ngather])
        gch = (pt_base // P) + ci
        pltpu.sync_copy(jidx_hbm.at[pl.ds(gch * nrows, nrows)], jv)
        pltpu.sync_copy(meta_hbm.at[pl.ds(gch * mstride, mstride)], m_v)
        handles = []
        for t in range(2):
            for g in range(ngather):
                handles.append(pltpu.async_copy(
                    cvt_hbm.at[jv.at[t * ngather + g]], rv[t][g],
                    semg.at[s, t]))
        return handles

    def compute(ci, s, handles):
        m_v, o_v = slots[s][1], slots[s][2 + 2 * ngather]
        rv = (slots[s][2:2 + ngather], slots[s][2 + ngather:2 + 2 * ngather])
        for hnd in handles:
            hnd.wait()
        for g in range(ngather):
            @pl.loop(0, IDXW, unroll=4)
            def _row(r, g=g):
                t = g * IDXW + r
                n = t // bands
                b = t % bands
                tv = plsc.load_gather(m_v,
                                      [jnp.full((16,), t * 3, jnp.int32)
                                       + lane])
                w0s = tv[0]
                w1s = tv[1]
                ys = tv[2]
                flat = jnp.full((16,), n * cols + b, jnp.int32)
                for k in range(c // 16):
                    gg0 = rv[0][g][r, pl.ds(k * 16, 16)]
                    gg1 = rv[1][g][r, pl.ds(k * 16, 16)]
                    val = gg0 * w0s + gg1 * w1s + ys
                    plsc.store_scatter(o_v, [flat + lane_cols[k]], val)

        pltpu.sync_copy(o_v,
                        out_hbm.at[pl.ds((pt_base + ci * P) * cols, P * cols)])

    @pl.loop(0, nchunks, step=NSLOT)
    def _pair(ci):
        hs = [stage(ci + s, s) for s in range(NSLOT)]
        for s in range(NSLOT):
            compute(ci + s, s, hs[s])


def kernel(points, scale, freqs, cv):
    n = points.shape[0]
    f = freqs.shape[0]
    bands = f * 2 * 3
    c = cv.shape[1]
    h = cv.shape[2]
    cols = bands * c
    assert n % (NUM_WORKERS * P) == 0 and c % 16 == 0 and (P * bands) % IDXW == 0

    fidx = np.arange(bands) // (2 * 3)
    dsel = np.arange(bands) % 3
    onehot = jnp.asarray((dsel[None, :] == np.arange(3)[:, None]).astype(np.float32))
    m = onehot * (freqs[fidx][None, :] / scale)

    cvt = pl.pallas_call(
        _transpose_body,
        grid=(bands,),
        in_specs=[pl.BlockSpec((1, c, h), lambda b: (b, 0, 0))],
        out_specs=pl.BlockSpec((h, c), lambda b: (b, 0)),
        out_shape=jax.ShapeDtypeStruct((bands * h, c), jnp.float32),
    )(cv.reshape(bands, c, h))

    nb = 2048
    j0, j1, w0, w1, y = pl.pallas_call(
        functools.partial(_meta_body, h, bands),
        grid=(n // nb,),
        in_specs=[pl.BlockSpec((nb, 3), lambda i: (i, 0)),
                  pl.BlockSpec((3, bands), lambda i: (0, 0))],
        out_specs=[pl.BlockSpec((nb, bands), lambda i: (i, 0))] * 5,
        out_shape=[jax.ShapeDtypeStruct((n, bands), jnp.int32),
                   jax.ShapeDtypeStruct((n, bands), jnp.int32),
                   jax.ShapeDtypeStruct((n, bands), jnp.float32),
                   jax.ShapeDtypeStruct((n, bands), jnp.float32),
                   jax.ShapeDtypeStruct((n, bands), jnp.float32)],
    )(points, m)

    npw = n // NUM_WORKERS
    nchunks = npw // P
    assert nchunks % NSLOT == 0
    ngather = (P * bands) // IDXW
    nrows = 2 * ngather + (-2 * ngather) % 8
    mesh = plsc.VectorSubcoreMesh(core_axis_name="c", subcore_axis_name="s")
    cp = pltpu.CompilerParams()
    if "needs_layout_passes" in pltpu.CompilerParams.__dataclass_fields__:
        cp = dataclasses.replace(cp, needs_layout_passes=False)
    if "use_tc_tiling_on_sc" in pltpu.CompilerParams.__dataclass_fields__:
        cp = dataclasses.replace(cp, use_tc_tiling_on_sc=False)
    tpc = P * bands
    sc = pl.kernel(
        functools.partial(_sc_body, npw, nchunks, bands, c),
        compiler_params=cp,
        out_type=jax.ShapeDtypeStruct((n * cols,), jnp.float32),
        mesh=mesh,
        scratch_types=(
            ([pltpu.VMEM((nrows, IDXW), jnp.int32),
              pltpu.VMEM((tpc * 3 + 16,), jnp.float32)]
             + [pltpu.VMEM((IDXW, c), jnp.float32)] * (2 * ngather)
             + [pltpu.VMEM((P * cols,), jnp.float32)]) * NSLOT
            + [pltpu.SemaphoreType.DMA((NSLOT, 2))]
        ),
    )
    nct = n // P
    jblk = jnp.concatenate(
        [j0.reshape(nct, ngather, IDXW), j1.reshape(nct, ngather, IDXW),
         jnp.zeros((nct, nrows - 2 * ngather, IDXW), jnp.int32)], axis=1)
    meta = jnp.stack([w0, w1, y], axis=-1).reshape(nct, tpc * 3)
    meta = jnp.pad(meta, ((0, 0), (0, 16))).reshape(-1)
    out = sc(cvt, jblk.reshape(nct * nrows, IDXW), meta)
    return out.reshape(n, cols)


# speedup vs baseline: 1.1131x; 1.0304x over previous
import dataclasses
import functools

import numpy as np
import jax
import jax.numpy as jnp
from jax import lax
from jax.experimental import pallas as pl
from jax.experimental.pallas import tpu as pltpu
from jax.experimental.pallas import tpu_sc as plsc

NUM_WORKERS = 32
P = 8
IDXW = 96
NSLOT = 2


def _transpose_body(cv_ref, out_ref):
    out_ref[...] = cv_ref[0].T


def _meta_body(h, bands, pts_ref, m_ref, j0_ref, j1_ref, w0_ref, w1_ref, y_ref):
    pts = pts_ref[...]
    m = m_ref[...]
    fp = (pts[:, 0:1] * m[0:1, :] + pts[:, 1:2] * m[1:2, :]
          + pts[:, 2:3] * m[2:3, :])
    nb = fp.shape[0]
    col = lax.broadcasted_iota(jnp.int32, (nb, bands), 1)
    is_sin = ((col // 3) % 2) == 0
    y = jnp.where(is_sin, jnp.sin(fp), jnp.cos(fp))
    iy = (y + 1.0) * ((h - 1) * 0.5)
    i0f = jnp.floor(iy)
    fr = iy - i0f
    i1f = i0f + 1.0
    v0 = ((i0f >= 0.0) & (i0f <= h - 1.0)).astype(jnp.float32)
    v1 = ((i1f >= 0.0) & (i1f <= h - 1.0)).astype(jnp.float32)
    i0 = jnp.clip(i0f, 0.0, h - 1.0).astype(jnp.int32)
    i1 = jnp.clip(i1f, 0.0, h - 1.0).astype(jnp.int32)
    j0_ref[...] = i0 + col * h
    j1_ref[...] = i1 + col * h
    w0_ref[...] = (1.0 - fr) * v0
    w1_ref[...] = fr * v1
    y_ref[...] = y


def _sc_body(npw, nchunks, bands, c, cvt_hbm, jidx_hbm, meta_hbm,
             out_hbm, *rest):
    ncores = 2
    ngather = (P * bands) // IDXW
    nrows = 2 * ngather + (-2 * ngather) % 8
    per = 2 + 2 * ngather + 1
    slots = [rest[s * per:(s + 1) * per] for s in range(NSLOT)]
    semg, semm, semo = rest[NSLOT * per:NSLOT * per + 3]
    wid = lax.axis_index("s") * ncores + lax.axis_index("c")
    pt_base = wid * npw
    cols = bands * c
    tpc = P * bands
    mstride = tpc * 3 + 16
    lane = lax.iota(jnp.int32, 16)
    lane_cols = [lane * bands + k * 16 * bands for k in range(c // 16)]

    def refs(s):
        jv, m_v = slots[s][0], slots[s][1]
        rv = (slots[s][2:2 + ngather], slots[s][2 + ngather:2 + 2 * ngather])
        o_v = slots[s][2 + 2 * ngather]
        return jv, m_v, rv, o_v

    def stage(ci, s):
        jv, m_v, rv, _ = refs(s)
        gch = (pt_base // P) + ci
        pltpu.sync_copy(jidx_hbm.at[pl.ds(gch * nrows, nrows)], jv)
        pltpu.async_copy(meta_hbm.at[pl.ds(gch * mstride, mstride)], m_v,
                         semm.at[s])
        for t in range(2):
            for g in range(ngather):
                pltpu.async_copy(cvt_hbm.at[jv.at[t * ngather + g]],
                                 rv[t][g], semg.at[s, t])

    def drain_out(s):
        _, _, _, o_v = refs(s)
        pltpu.make_async_copy(o_v, out_hbm.at[pl.ds(0, P * cols)],
                              semo.at[s]).wait()

    def compute(ci, s, drain):
        jv, m_v, rv, o_v = refs(s)
        for t in range(2):
            for g in range(ngather):
                pltpu.make_async_copy(cvt_hbm.at[jv.at[t * ngather + g]],
                                      rv[t][g], semg.at[s, t]).wait()
        pltpu.make_async_copy(meta_hbm.at[pl.ds(0, mstride)], m_v,
                              semm.at[s]).wait()
        if drain:
            drain_out(s)
        for g in range(ngather):
            @pl.loop(0, IDXW, unroll=4)
            def _row(r, g=g):
                t = g * IDXW + r
                n = t // bands
                b = t % bands
                tv = plsc.load_gather(m_v,
                                      [jnp.full((16,), t * 3, jnp.int32)
                                       + lane])
                w0s = tv[0]
                w1s = tv[1]
                ys = tv[2]
                flat = jnp.full((16,), n * cols + b, jnp.int32)
                for k in range(c // 16):
                    gg0 = rv[0][g][r, pl.ds(k * 16, 16)]
                    gg1 = rv[1][g][r, pl.ds(k * 16, 16)]
                    val = gg0 * w0s + gg1 * w1s + ys
                    plsc.store_scatter(o_v, [flat + lane_cols[k]], val)

        pltpu.async_copy(o_v,
                         out_hbm.at[pl.ds((pt_base + ci * P) * cols,
                                          P * cols)], semo.at[s])

    stage(0, 0)
    stage(1, 1)
    compute(0, 0, drain=False)
    stage(2, 0)
    compute(1, 1, drain=False)
    stage(3, 1)

    @pl.loop(2, nchunks - 2, step=2)
    def _main(i):
        compute(i, 0, drain=True)
        stage(i + 2, 0)
        compute(i + 1, 1, drain=True)
        stage(i + 3, 1)

    compute(nchunks - 2, 0, drain=True)
    compute(nchunks - 1, 1, drain=True)
    drain_out(0)
    drain_out(1)


def kernel(points, scale, freqs, cv):
    n = points.shape[0]
    f = freqs.shape[0]
    bands = f * 2 * 3
    c = cv.shape[1]
    h = cv.shape[2]
    cols = bands * c
    assert n % (NUM_WORKERS * P) == 0 and c % 16 == 0 and (P * bands) % IDXW == 0

    fidx = np.arange(bands) // (2 * 3)
    dsel = np.arange(bands) % 3
    onehot = jnp.asarray((dsel[None, :] == np.arange(3)[:, None]).astype(np.float32))
    m = onehot * (freqs[fidx][None, :] / scale)

    cvt = pl.pallas_call(
        _transpose_body,
        grid=(bands,),
        in_specs=[pl.BlockSpec((1, c, h), lambda b: (b, 0, 0))],
        out_specs=pl.BlockSpec((h, c), lambda b: (b, 0)),
        out_shape=jax.ShapeDtypeStruct((bands * h, c), jnp.float32),
    )(cv.reshape(bands, c, h))

    nb = 2048
    j0, j1, w0, w1, y = pl.pallas_call(
        functools.partial(_meta_body, h, bands),
        grid=(n // nb,),
        in_specs=[pl.BlockSpec((nb, 3), lambda i: (i, 0)),
                  pl.BlockSpec((3, bands), lambda i: (0, 0))],
        out_specs=[pl.BlockSpec((nb, bands), lambda i: (i, 0))] * 5,
        out_shape=[jax.ShapeDtypeStruct((n, bands), jnp.int32),
                   jax.ShapeDtypeStruct((n, bands), jnp.int32),
                   jax.ShapeDtypeStruct((n, bands), jnp.float32),
                   jax.ShapeDtypeStruct((n, bands), jnp.float32),
                   jax.ShapeDtypeStruct((n, bands), jnp.float32)],
    )(points, m)

    npw = n // NUM_WORKERS
    nchunks = npw // P
    assert nchunks % NSLOT == 0
    ngather = (P * bands) // IDXW
    nrows = 2 * ngather + (-2 * ngather) % 8
    mesh = plsc.VectorSubcoreMesh(core_axis_name="c", subcore_axis_name="s")
    cp = pltpu.CompilerParams()
    if "needs_layout_passes" in pltpu.CompilerParams.__dataclass_fields__:
        cp = dataclasses.replace(cp, needs_layout_passes=False)
    if "use_tc_tiling_on_sc" in pltpu.CompilerParams.__dataclass_fields__:
        cp = dataclasses.replace(cp, use_tc_tiling_on_sc=False)
    tpc = P * bands
    sc = pl.kernel(
        functools.partial(_sc_body, npw, nchunks, bands, c),
        compiler_params=cp,
        out_type=jax.ShapeDtypeStruct((n * cols,), jnp.float32),
        mesh=mesh,
        scratch_types=(
            ([pltpu.VMEM((nrows, IDXW), jnp.int32),
              pltpu.VMEM((tpc * 3 + 16,), jnp.float32)]
             + [pltpu.VMEM((IDXW, c), jnp.float32)] * (2 * ngather)
             + [pltpu.VMEM((P * cols,), jnp.float32)]) * NSLOT
            + [pltpu.SemaphoreType.DMA((NSLOT, 2)),
               pltpu.SemaphoreType.DMA((NSLOT,)),
               pltpu.SemaphoreType.DMA((NSLOT,))]
        ),
    )
    nct = n // P
    jblk = jnp.concatenate(
        [j0.reshape(nct, ngather, IDXW), j1.reshape(nct, ngather, IDXW),
         jnp.zeros((nct, nrows - 2 * ngather, IDXW), jnp.int32)], axis=1)
    meta = jnp.stack([w0, w1, y], axis=-1).reshape(nct, tpc * 3)
    meta = jnp.pad(meta, ((0, 0), (0, 16))).reshape(-1)
    out = sc(cvt, jblk.reshape(nct * nrows, IDXW), meta)
    return out.reshape(n, cols)


# speedup vs baseline: 1.1161x; 1.0027x over previous
import dataclasses
import functools

import numpy as np
import jax
import jax.numpy as jnp
from jax import lax
from jax.experimental import pallas as pl
from jax.experimental.pallas import tpu as pltpu
from jax.experimental.pallas import tpu_sc as plsc

NUM_WORKERS = 32
P = 8
IDXW = 96
NSLOT = 2


def _transpose_body(cv_ref, out_ref):
    out_ref[...] = cv_ref[0].T


def _meta_body(h, bands, pts_ref, m_ref, j0_ref, j1_ref, w0_ref, w1_ref, y_ref):
    pts = pts_ref[...]
    m = m_ref[...]
    fp = (pts[:, 0:1] * m[0:1, :] + pts[:, 1:2] * m[1:2, :]
          + pts[:, 2:3] * m[2:3, :])
    nb = fp.shape[0]
    col = lax.broadcasted_iota(jnp.int32, (nb, bands), 1)
    is_sin = ((col // 3) % 2) == 0
    y = jnp.where(is_sin, jnp.sin(fp), jnp.cos(fp))
    iy = (y + 1.0) * ((h - 1) * 0.5)
    i0f = jnp.floor(iy)
    fr = iy - i0f
    i1f = i0f + 1.0
    v0 = ((i0f >= 0.0) & (i0f <= h - 1.0)).astype(jnp.float32)
    v1 = ((i1f >= 0.0) & (i1f <= h - 1.0)).astype(jnp.float32)
    i0 = jnp.clip(i0f, 0.0, h - 1.0).astype(jnp.int32)
    i1 = jnp.clip(i1f, 0.0, h - 1.0).astype(jnp.int32)
    j0_ref[...] = i0 + col * h
    j1_ref[...] = i1 + col * h
    w0_ref[...] = (1.0 - fr) * v0
    w1_ref[...] = fr * v1
    y_ref[...] = y


def _sc_body(npw, nchunks, bands, c, cvt_hbm, jidx_hbm, meta_hbm,
             out_hbm, *rest):
    ncores = 2
    ngather = (P * bands) // IDXW
    nrows = 2 * ngather + (-2 * ngather) % 8
    per = 2 + 2 * ngather + 1
    slots = [rest[s * per:(s + 1) * per] for s in range(NSLOT)]
    semg, semm, semo = rest[NSLOT * per:NSLOT * per + 3]
    wid = lax.axis_index("s") * ncores + lax.axis_index("c")
    pt_base = wid * npw
    cols = bands * c
    tpc = P * bands
    mstride = tpc * 3 + 16
    lane = lax.iota(jnp.int32, 16)
    lane_cols = [lane * bands + k * 16 * bands for k in range(c // 16)]

    def refs(s):
        jv, m_v = slots[s][0], slots[s][1]
        rv = (slots[s][2:2 + ngather], slots[s][2 + ngather:2 + 2 * ngather])
        o_v = slots[s][2 + 2 * ngather]
        return jv, m_v, rv, o_v

    def stage(ci, s):
        jv, m_v, rv, _ = refs(s)
        gch = (pt_base // P) + ci
        pltpu.sync_copy(jidx_hbm.at[pl.ds(gch * nrows, nrows)], jv)
        pltpu.async_copy(meta_hbm.at[pl.ds(gch * mstride, mstride)], m_v,
                         semm.at[s])
        for t in range(2):
            for g in range(ngather):
                pltpu.async_copy(cvt_hbm.at[jv.at[t * ngather + g]],
                                 rv[t][g], semg.at[s, t])

    def drain_out(s):
        _, _, _, o_v = refs(s)
        pltpu.make_async_copy(o_v, out_hbm.at[pl.ds(0, P * cols)],
                              semo.at[s]).wait()

    def compute(ci, s, drain):
        jv, m_v, rv, o_v = refs(s)
        for t in range(2):
            for g in range(ngather):
                pltpu.make_async_copy(cvt_hbm.at[jv.at[t * ngather + g]],
                                      rv[t][g], semg.at[s, t]).wait()
        pltpu.make_async_copy(meta_hbm.at[pl.ds(0, mstride)], m_v,
                              semm.at[s]).wait()
        if drain:
            drain_out(s)
        for g in range(ngather):
            @pl.loop(0, IDXW, unroll=8)
            def _row(r, g=g):
                t = g * IDXW + r
                n = t // bands
                b = t % bands
                tv = plsc.load_gather(m_v,
                                      [jnp.full((16,), t * 3, jnp.int32)
                                       + lane])
                w0s = tv[0]
                w1s = tv[1]
                ys = tv[2]
                flat = jnp.full((16,), n * cols + b, jnp.int32)
                for k in range(c // 16):
                    gg0 = rv[0][g][r, pl.ds(k * 16, 16)]
                    gg1 = rv[1][g][r, pl.ds(k * 16, 16)]
                    val = gg0 * w0s + gg1 * w1s + ys
                    plsc.store_scatter(o_v, [flat + lane_cols[k]], val)

        pltpu.async_copy(o_v,
                         out_hbm.at[pl.ds((pt_base + ci * P) * cols,
                                          P * cols)], semo.at[s])

    stage(0, 0)
    stage(1, 1)
    compute(0, 0, drain=False)
    stage(2, 0)
    compute(1, 1, drain=False)
    stage(3, 1)

    @pl.loop(2, nchunks - 2, step=2)
    def _main(i):
        compute(i, 0, drain=True)
        stage(i + 2, 0)
        compute(i + 1, 1, drain=True)
        stage(i + 3, 1)

    compute(nchunks - 2, 0, drain=True)
    compute(nchunks - 1, 1, drain=True)
    drain_out(0)
    drain_out(1)


def kernel(points, scale, freqs, cv):
    n = points.shape[0]
    f = freqs.shape[0]
    bands = f * 2 * 3
    c = cv.shape[1]
    h = cv.shape[2]
    cols = bands * c
    assert n % (NUM_WORKERS * P) == 0 and c % 16 == 0 and (P * bands) % IDXW == 0

    fidx = np.arange(bands) // (2 * 3)
    dsel = np.arange(bands) % 3
    onehot = jnp.asarray((dsel[None, :] == np.arange(3)[:, None]).astype(np.float32))
    m = onehot * (freqs[fidx][None, :] / scale)

    cvt = pl.pallas_call(
        _transpose_body,
        grid=(bands,),
        in_specs=[pl.BlockSpec((1, c, h), lambda b: (b, 0, 0))],
        out_specs=pl.BlockSpec((h, c), lambda b: (b, 0)),
        out_shape=jax.ShapeDtypeStruct((bands * h, c), jnp.float32),
    )(cv.reshape(bands, c, h))

    nb = 2048
    j0, j1, w0, w1, y = pl.pallas_call(
        functools.partial(_meta_body, h, bands),
        grid=(n // nb,),
        in_specs=[pl.BlockSpec((nb, 3), lambda i: (i, 0)),
                  pl.BlockSpec((3, bands), lambda i: (0, 0))],
        out_specs=[pl.BlockSpec((nb, bands), lambda i: (i, 0))] * 5,
        out_shape=[jax.ShapeDtypeStruct((n, bands), jnp.int32),
                   jax.ShapeDtypeStruct((n, bands), jnp.int32),
                   jax.ShapeDtypeStruct((n, bands), jnp.float32),
                   jax.ShapeDtypeStruct((n, bands), jnp.float32),
                   jax.ShapeDtypeStruct((n, bands), jnp.float32)],
    )(points, m)

    npw = n // NUM_WORKERS
    nchunks = npw // P
    assert nchunks % NSLOT == 0
    ngather = (P * bands) // IDXW
    nrows = 2 * ngather + (-2 * ngather) % 8
    mesh = plsc.VectorSubcoreMesh(core_axis_name="c", subcore_axis_name="s")
    cp = pltpu.CompilerParams()
    if "needs_layout_passes" in pltpu.CompilerParams.__dataclass_fields__:
        cp = dataclasses.replace(cp, needs_layout_passes=False)
    if "use_tc_tiling_on_sc" in pltpu.CompilerParams.__dataclass_fields__:
        cp = dataclasses.replace(cp, use_tc_tiling_on_sc=False)
    tpc = P * bands
    sc = pl.kernel(
        functools.partial(_sc_body, npw, nchunks, bands, c),
        compiler_params=cp,
        out_type=jax.ShapeDtypeStruct((n * cols,), jnp.float32),
        mesh=mesh,
        scratch_types=(
            ([pltpu.VMEM((nrows, IDXW), jnp.int32),
              pltpu.VMEM((tpc * 3 + 16,), jnp.float32)]
             + [pltpu.VMEM((IDXW, c), jnp.float32)] * (2 * ngather)
             + [pltpu.VMEM((P * cols,), jnp.float32)]) * NSLOT
            + [pltpu.SemaphoreType.DMA((NSLOT, 2)),
               pltpu.SemaphoreType.DMA((NSLOT,)),
               pltpu.SemaphoreType.DMA((NSLOT,))]
        ),
    )
    nct = n // P
    jblk = jnp.concatenate(
        [j0.reshape(nct, ngather, IDXW), j1.reshape(nct, ngather, IDXW),
         jnp.zeros((nct, nrows - 2 * ngather, IDXW), jnp.int32)], axis=1)
    meta = jnp.stack([w0, w1, y], axis=-1).reshape(nct, tpc * 3)
    meta = jnp.pad(meta, ((0, 0), (0, 16))).reshape(-1)
    out = sc(cvt, jblk.reshape(nct * nrows, IDXW), meta)
    return out.reshape(n, cols)
